# Initial kernel scaffold; baseline (speedup 1.0000x reference)
import jax
import jax.numpy as jnp
from jax.experimental import pallas as pl


def kernel(x):
    raise NotImplementedError("write your pallas kernel here")


# speedup vs baseline: 1.5816x; 1.5816x over previous
import jax
import jax.numpy as jnp
from jax.experimental import pallas as pl
from jax.experimental.pallas import tpu as pltpu

NUM_FIELDS = 26
CARD = 100
OUT_D = NUM_FIELDS * CARD
ROWS = 16384
BLOCK_ROWS = 256


def _onehot_block(x_ref, o_ref):
    iota = jax.lax.broadcasted_iota(jnp.int32, (BLOCK_ROWS, CARD), 1)
    for i in range(NUM_FIELDS):
        v = x_ref[:, i][:, None]
        o_ref[:, i * CARD:(i + 1) * CARD] = (iota == v).astype(jnp.float32)


def kernel(x):
    return pl.pallas_call(
        _onehot_block,
        grid=(ROWS // BLOCK_ROWS,),
        in_specs=[pl.BlockSpec((BLOCK_ROWS, NUM_FIELDS), lambda r: (r, 0))],
        out_specs=pl.BlockSpec((BLOCK_ROWS, OUT_D), lambda r: (r, 0)),
        out_shape=jax.ShapeDtypeStruct((ROWS, OUT_D), jnp.float32),
    )(x)


# speedup vs baseline: 9.8825x; 6.2482x over previous
import jax
import jax.numpy as jnp
from jax.experimental import pallas as pl

NUM_FIELDS = 26
CARD = 100
OUT_D = NUM_FIELDS * CARD
ROWS = 16384
BB = 512


def _onehot_t_block(xt_ref, o_ref):
    iota2 = jax.lax.broadcasted_iota(jnp.int32, (2 * CARD, BB), 0)
    hi = iota2 >= CARD
    mod = iota2 - jnp.where(hi, CARD, 0)
    for k in range(NUM_FIELDS // 2):
        v0 = xt_ref[2 * k, :][None, :]
        v1 = xt_ref[2 * k + 1, :][None, :]
        v = jnp.where(hi, v1, v0)
        o_ref[2 * CARD * k:2 * CARD * (k + 1), :] = (mod == v).astype(jnp.float32)


def kernel(x):
    xt = x.T
    out_t = pl.pallas_call(
        _onehot_t_block,
        grid=(ROWS // BB,),
        in_specs=[pl.BlockSpec((NUM_FIELDS, BB), lambda j: (0, j))],
        out_specs=pl.BlockSpec((OUT_D, BB), lambda j: (0, j)),
        out_shape=jax.ShapeDtypeStruct((OUT_D, ROWS), jnp.float32),
    )(xt)
    return out_t.T
